# Initial kernel scaffold; baseline (speedup 1.0000x reference)
import jax
import jax.numpy as jnp
from jax.experimental import pallas as pl


def kernel(output, target, anchors):
    raise NotImplementedError("write your pallas kernel here")


# speedup vs baseline: 16.3870x; 16.3870x over previous
import jax
import jax.numpy as jnp
from jax import lax
from jax.experimental import pallas as pl
from jax.experimental.pallas import tpu as pltpu

_NC = 8
_NA = 5
_NH = 48
_NW = 48
_NB = 8
_NT = 50
_SIL = 0.6
_ROWS = 18
_LANES = 128
_INTERPRET = False


def _table_body(tg_ref, anc_ref, tab_ref):
    f32 = jnp.float32
    tcls = tg_ref[0]
    xc = tg_ref[1]
    gx = xc * _NW
    gy = tg_ref[2] * _NH
    gw = tg_ref[3] * _NW
    gl = tg_ref[4] * _NH
    tim = tg_ref[5]
    tre = tg_ref[6]
    bad = (xc == 0.0).astype(f32)
    r = lax.broadcasted_iota(jnp.int32, (_NT, _NT), 0)
    c = lax.broadcasted_iota(jnp.int32, (_NT, _NT), 1)
    tri = (c <= r).astype(f32)
    cnt = jnp.sum(bad[:, None, :] * tri[None, :, :], axis=2)
    validf = cnt == 0.0
    garea = gw * gl
    best_iou = jnp.zeros_like(gw)
    best = jnp.zeros_like(gw)
    awb = jnp.zeros_like(gw)
    ahb = jnp.zeros_like(gw)
    for k in range(_NA):
        aw = anc_ref[2 * k]
        ah = anc_ref[2 * k + 1]
        cw = jnp.minimum(aw, gw)
        ch = jnp.minimum(ah, gl)
        carea = jnp.where((cw <= 0.0) | (ch <= 0.0), 0.0, cw * ch)
        iou = carea / (aw * ah + garea - carea)
        upd = iou > best_iou
        best = jnp.where(upd, float(k), best)
        awb = jnp.where(upd, aw, awb)
        ahb = jnp.where(upd, ah, ahb)
        best_iou = jnp.maximum(best_iou, iou)
    neg = best_iou <= 0.0
    nmod = jnp.where(neg, 4.0, best)
    awsel = jnp.where(neg, anc_ref[8], awb)
    ahsel = jnp.where(neg, anc_ref[9], ahb)
    gi = jnp.floor(gx)
    gj = jnp.floor(gy)
    tab_ref[0] = jnp.where(validf, nmod, -1.0)
    tab_ref[1] = gj * _NW + gi
    tab_ref[2] = gx - 0.5 * gw
    tab_ref[3] = gx + 0.5 * gw
    tab_ref[4] = gy - 0.5 * gl
    tab_ref[5] = gy + 0.5 * gl
    tab_ref[6] = gw
    tab_ref[7] = gl
    tab_ref[8] = garea
    tab_ref[9] = gx - gi
    tab_ref[10] = gy - gj
    tab_ref[11] = jnp.log(gw / awsel)
    tab_ref[12] = jnp.log(gl / ahsel)
    tab_ref[13] = tim
    tab_ref[14] = tre
    tab_ref[15] = tcls


def _loss_body(o_ref, tab_ref, anc_ref, out_ref):
    f32 = jnp.float32
    step = pl.program_id(0)
    b = step // _NA
    a = step % _NA
    af = a.astype(f32)
    base = b * _NT
    x = jax.nn.sigmoid(o_ref[0])
    y = jax.nn.sigmoid(o_ref[1])
    w = o_ref[2]
    ll = o_ref[3]
    im = o_ref[4]
    re = o_ref[5]
    conf = jax.nn.sigmoid(o_ref[6])
    aw = anc_ref[2 * a]
    ah = anc_ref[2 * a + 1]
    ri = lax.broadcasted_iota(jnp.int32, (_ROWS, _LANES), 0)
    ci = lax.broadcasted_iota(jnp.int32, (_ROWS, _LANES), 1)
    lin = (ri * _LANES + ci).astype(f32)
    fj = jnp.floor(lin * (1.0 / _NW))
    fi = lin - fj * _NW
    px = x + fi
    py = y + fj
    pw = jnp.exp(w) * aw
    pll = jnp.exp(ll) * ah
    pxl = px - 0.5 * pw
    pxh = px + 0.5 * pw
    pyl = py - 0.5 * pll
    pyh = py + 0.5 * pll
    parea = pw * pll

    def body1(t, carry):
        cur, wt, wiou = carry
        idx = base + t
        code = tab_ref[0, idx]
        key = tab_ref[1, idx]
        gxl = tab_ref[2, idx]
        gxh = tab_ref[3, idx]
        gyl = tab_ref[4, idx]
        gyh = tab_ref[5, idx]
        gw = tab_ref[6, idx]
        gl = tab_ref[7, idx]
        garea = tab_ref[8, idx]
        valid = code > -0.5
        uw = jnp.maximum(pxh, gxh) - jnp.minimum(pxl, gxl)
        uh = jnp.maximum(pyh, gyh) - jnp.minimum(pyl, gyl)
        cw = pw + gw - uw
        ch = pll + gl - uh
        carea = jnp.where((cw <= 0.0) | (ch <= 0.0), 0.0, cw * ch)
        iou = carea / (parea + garea - carea)
        iou = jnp.where(valid, iou, 0.0)
        cur = jnp.maximum(cur, iou)
        match = jnp.logical_and(valid & (code == af), lin == key)
        wt = jnp.where(match, t.astype(f32), wt)
        wiou = jnp.where(match, iou, wiou)
        return cur, wt, wiou

    z = jnp.zeros((_ROWS, _LANES), f32)
    cur, wt, wiou = lax.fori_loop(0, _NT, body1, (z, z - 1.0, z))

    def body2(t, vals):
        idx = base + t
        match = wt == t.astype(f32)
        return tuple(
            jnp.where(match, tab_ref[9 + s, idx], v) for s, v in enumerate(vals)
        )

    vtx, vty, vtw, vtl, vtim, vtre, vtcls = lax.fori_loop(
        0, _NT, body2, (z, z, z, z, z, z, z))

    has = wt > -0.5
    coord = ((x - vtx) ** 2 + (y - vty) ** 2 + (w - vtw) ** 2 + (ll - vtl) ** 2
             + (im - vtim) ** 2 + (re - vtre) ** 2)
    coord = jnp.where(has, coord, 0.0)
    confterm = jnp.where(has, 100.0 * (conf - wiou) ** 2,
                         jnp.where(cur > _SIL, 0.0, conf * conf))
    cls = o_ref[7:7 + _NC]
    m = jnp.max(cls, axis=0)
    lse = m + jnp.log(jnp.sum(jnp.exp(cls - m[None]), axis=0))
    lab = jnp.floor(vtcls)
    picked = z
    for cc in range(_NC):
        picked = jnp.where(lab == float(cc), cls[cc], picked)
    clsterm = jnp.where(has, lse - picked, 0.0)
    total = jnp.sum(0.5 * (coord + confterm) + clsterm)

    @pl.when(step == 0)
    def _():
        out_ref[0, 0] = 0.0

    out_ref[0, 0] += total


def kernel(output, target, anchors):
    tgt_t = jnp.transpose(target, (2, 0, 1)).astype(jnp.float32)
    tab = pl.pallas_call(
        _table_body,
        out_shape=jax.ShapeDtypeStruct((16, _NB, _NT), jnp.float32),
        in_specs=[
            pl.BlockSpec(memory_space=pltpu.VMEM),
            pl.BlockSpec(memory_space=pltpu.SMEM),
        ],
        out_specs=pl.BlockSpec(memory_space=pltpu.VMEM),
        interpret=_INTERPRET,
    )(tgt_t, anchors)
    tab2 = tab.reshape(16, _NB * _NT)
    o3 = output.reshape(_NB * _NA * (7 + _NC), _ROWS, _LANES)
    res = pl.pallas_call(
        _loss_body,
        grid=(_NB * _NA,),
        in_specs=[
            pl.BlockSpec((7 + _NC, _ROWS, _LANES), lambda i: (i, 0, 0)),
            pl.BlockSpec(memory_space=pltpu.SMEM),
            pl.BlockSpec(memory_space=pltpu.SMEM),
        ],
        out_specs=pl.BlockSpec(memory_space=pltpu.SMEM),
        out_shape=jax.ShapeDtypeStruct((1, 1), jnp.float32),
        interpret=_INTERPRET,
    )(o3, tab2, anchors)
    return res[0, 0]
